# baseline (speedup 1.0000x reference)
import jax
import jax.numpy as jnp
from jax import lax
from jax.experimental import pallas as pl
from jax.experimental.pallas import tpu as pltpu
from jax.experimental.pallas import tpu_sc as plsc

_DEPTH = 14
_KTC = 8
_NSH = 2 ** _KTC
_L = 16
_BT = 2048


def _tc_phase(x, xt, ysh):
    n_batch, d_in = x.shape
    d_out = ysh.shape[-1]
    grid = n_batch // _BT

    def body(x_ref, xt_ref, ysh_ref, ypart_ref, node_ref):
        xb = x_ref[...]
        p_mat = jnp.dot(xb, xt_ref[...],
                        preferred_element_type=jnp.float32,
                        precision=lax.Precision.HIGHEST)
        ids = lax.broadcasted_iota(jnp.int32, (_BT, _NSH), 1)
        p = jnp.zeros((_BT, 1), jnp.int32)
        w = jnp.zeros((_BT, _NSH), jnp.float32)
        for d in range(_KTC):
            col = (2 ** d - 1) + p
            m = ids == col
            lam = jnp.sum(jnp.where(m, p_mat, 0.0), axis=1, keepdims=True)
            w = w + jnp.where(m, lam, 0.0)
            p = p + p + (lam > 0.0).astype(jnp.int32)
        ypart_ref[...] = jnp.dot(w, ysh_ref[...],
                                 preferred_element_type=jnp.float32)
        node_ref[0, 0, :] = (_NSH - 1) + p[:, 0]

    return pl.pallas_call(
        body,
        grid=(grid,),
        in_specs=[
            pl.BlockSpec((_BT, d_in), lambda i: (i, 0)),
            pl.BlockSpec((d_in, _NSH), lambda i: (0, 0)),
            pl.BlockSpec((_NSH, d_out), lambda i: (0, 0)),
        ],
        out_specs=[
            pl.BlockSpec((_BT, d_out), lambda i: (i, 0)),
            pl.BlockSpec((1, 1, _BT), lambda i: (i, 0, 0)),
        ],
        out_shape=[
            jax.ShapeDtypeStruct((n_batch, d_out), jnp.float32),
            jax.ShapeDtypeStruct((grid, 1, _BT), jnp.int32),
        ],
    )(x, xt, ysh)


def _sc_phase(x, X, Y, y_part, node_init):
    n_batch, d_in = x.shape
    d_out = Y.shape[-1]
    info = plsc.get_sparse_core_info()
    nc, ns = info.num_cores, info.num_subcores
    nw = nc * ns
    b_per_w = n_batch // nw
    n_sub = b_per_w // _L
    n_deep = _DEPTH - _KTC

    def body(x_hbm, X_hbm, Y_hbm, ypart_hbm, node_hbm, out_hbm,
             x_v, y_v, xr_a, yr_a, xr_b, yr_b, node_r, node_nx, accm_r,
             sem_xa, sem_ya, sem_xb, sem_yb, sem_p1, sem_p2, sem_n, sem_w):
        wid = lax.axis_index("s") * nc + lax.axis_index("c")
        base = wid * b_per_w
        ids = lax.iota(jnp.int32, _L)
        bufs = [(xr_a, yr_a, sem_xa, sem_ya), (xr_b, yr_b, sem_xb, sem_yb)]

        pltpu.sync_copy(node_hbm.at[pl.ds(base, _L)], node_r)
        pltpu.async_copy(X_hbm.at[node_r], xr_a, sem_xa)
        pltpu.async_copy(Y_hbm.at[node_r], yr_a, sem_ya)

        def sub_chunk(sc_i, carry):
            tok0 = base + sc_i * _L

            @pl.when(sc_i > 0)
            def _():
                pltpu.make_async_copy(
                    y_v, out_hbm.at[pl.ds(tok0, _L)], sem_w).wait()

            @pl.when(sc_i == 0)
            def _():
                pltpu.async_copy(x_hbm.at[pl.ds(tok0, _L)], x_v, sem_p1)

            cp_yv = pltpu.async_copy(ypart_hbm.at[pl.ds(tok0, _L)], y_v,
                                     sem_p2)

            @pl.when(sc_i < n_sub - 1)
            def _():
                pltpu.async_copy(node_hbm.at[pl.ds(tok0 + _L, _L)],
                                 node_nx, sem_n)

            pltpu.make_async_copy(
                x_hbm.at[pl.ds(tok0, _L)], x_v, sem_p1).wait()
            for d in range(n_deep):
                xr_v, yr_v, sx, sy = bufs[d % 2]
                pltpu.make_async_copy(X_hbm.at[node_r], xr_v, sx).wait()

                def dot_body(i, accs):
                    sl = pl.ds(i * _L, _L)
                    return tuple(accs[t] + x_v[t, sl] * xr_v[t, sl]
                                 for t in range(_L))

                accs = lax.fori_loop(
                    0, d_in // _L, dot_body,
                    (jnp.zeros((_L,), jnp.float32),) * _L)
                for t in range(_L):
                    accm_r[pl.ds(t * _L, _L)] = accs[t]
                lam_vec = jnp.zeros((_L,), jnp.float32)
                for i in range(_L):
                    lam_vec = lam_vec + plsc.load_gather(
                        accm_r, [ids * _L + i])
                node = node_r[:]
                node_r[:] = node + node + 1 + (lam_vec > 0.0).astype(
                    jnp.int32)
                if d + 1 < n_deep:
                    nxr, nyr, nsx, nsy = bufs[(d + 1) % 2]
                    pltpu.async_copy(X_hbm.at[node_r], nxr, nsx)
                    pltpu.async_copy(Y_hbm.at[node_r], nyr, nsy)
                else:
                    @pl.when(sc_i < n_sub - 1)
                    def _():
                        pltpu.make_async_copy(
                            node_hbm.at[pl.ds(tok0 + _L, _L)],
                            node_nx, sem_n).wait()
                        node_r[:] = node_nx[:]
                        pltpu.async_copy(X_hbm.at[node_r], xr_a, sem_xa)
                        pltpu.async_copy(Y_hbm.at[node_r], yr_a, sem_ya)
                        pltpu.async_copy(x_hbm.at[pl.ds(tok0 + _L, _L)],
                                         x_v, sem_p1)

                lam_bs = [jnp.broadcast_to(lam_vec[t], (_L,))
                          for t in range(_L)]
                if d == 0:
                    cp_yv.wait()
                pltpu.make_async_copy(Y_hbm.at[node_r], yr_v, sy).wait()

                def acc_body(i, c2):
                    sl = pl.ds(i * _L, _L)
                    for t in range(_L):
                        y_v[t, sl] = y_v[t, sl] + lam_bs[t] * yr_v[t, sl]
                    return c2

                lax.fori_loop(0, d_out // _L, acc_body, 0)
            pltpu.async_copy(y_v, out_hbm.at[pl.ds(tok0, _L)], sem_w)
            return carry

        lax.fori_loop(0, n_sub, sub_chunk, 0)
        pltpu.make_async_copy(
            y_v, out_hbm.at[pl.ds(base + (n_sub - 1) * _L, _L)],
            sem_w).wait()

    fff = pl.kernel(
        body,
        out_type=jax.ShapeDtypeStruct((n_batch, d_out), jnp.float32),
        mesh=plsc.VectorSubcoreMesh(core_axis_name="c", subcore_axis_name="s"),
        compiler_params=pltpu.CompilerParams(needs_layout_passes=False),
        scratch_types=[
            pltpu.VMEM((_L, d_in), jnp.float32),
            pltpu.VMEM((_L, d_out), jnp.float32),
            pltpu.VMEM((_L, d_in), jnp.float32),
            pltpu.VMEM((_L, d_out), jnp.float32),
            pltpu.VMEM((_L, d_in), jnp.float32),
            pltpu.VMEM((_L, d_out), jnp.float32),
            pltpu.VMEM((_L,), jnp.int32),
            pltpu.VMEM((_L,), jnp.int32),
            pltpu.VMEM((_L * _L,), jnp.float32),
            pltpu.SemaphoreType.DMA,
            pltpu.SemaphoreType.DMA,
            pltpu.SemaphoreType.DMA,
            pltpu.SemaphoreType.DMA,
            pltpu.SemaphoreType.DMA,
            pltpu.SemaphoreType.DMA,
            pltpu.SemaphoreType.DMA,
            pltpu.SemaphoreType.DMA,
        ],
    )
    return fff(x, X, Y, y_part, node_init)


def kernel(x, X, Y):
    n_batch = x.shape[0]
    xt = jnp.swapaxes(X[:_NSH], 0, 1)
    ysh = Y[:_NSH]
    y_part, node3d = _tc_phase(x, xt, ysh)
    node_init = jnp.reshape(node3d, (n_batch,))
    return _sc_phase(x, X, Y, y_part, node_init)


# speedup vs baseline: 1.0022x; 1.0022x over previous
import jax
import jax.numpy as jnp
from jax import lax
from jax.experimental import pallas as pl
from jax.experimental.pallas import tpu as pltpu
from jax.experimental.pallas import tpu_sc as plsc

_DEPTH = 14
_KTC = 8
_NSH = 2 ** _KTC
_L = 16
_BT = 1024


def _tc_phase(x, xt, ysh):
    n_batch, d_in = x.shape
    d_out = ysh.shape[-1]
    grid = n_batch // _BT

    def body(x_ref, xt_ref, ysh_ref, ypart_ref, node_ref):
        xb = x_ref[...]
        p_mat = jnp.dot(xb, xt_ref[...],
                        preferred_element_type=jnp.float32,
                        precision=lax.Precision.HIGHEST)
        ids = lax.broadcasted_iota(jnp.int32, (_BT, _NSH), 1)
        p = jnp.zeros((_BT, 1), jnp.int32)
        w = jnp.zeros((_BT, _NSH), jnp.float32)
        for d in range(_KTC):
            col = (2 ** d - 1) + p
            m = ids == col
            lam = jnp.sum(jnp.where(m, p_mat, 0.0), axis=1, keepdims=True)
            w = w + jnp.where(m, lam, 0.0)
            p = p + p + (lam > 0.0).astype(jnp.int32)
        ypart_ref[...] = jnp.dot(w, ysh_ref[...],
                                 preferred_element_type=jnp.float32)
        node_ref[0, 0, :] = (_NSH - 1) + p[:, 0]

    return pl.pallas_call(
        body,
        grid=(grid,),
        in_specs=[
            pl.BlockSpec((_BT, d_in), lambda i: (i, 0)),
            pl.BlockSpec((d_in, _NSH), lambda i: (0, 0)),
            pl.BlockSpec((_NSH, d_out), lambda i: (0, 0)),
        ],
        out_specs=[
            pl.BlockSpec((_BT, d_out), lambda i: (i, 0)),
            pl.BlockSpec((1, 1, _BT), lambda i: (i, 0, 0)),
        ],
        out_shape=[
            jax.ShapeDtypeStruct((n_batch, d_out), jnp.float32),
            jax.ShapeDtypeStruct((grid, 1, _BT), jnp.int32),
        ],
    )(x, xt, ysh)


def _sc_phase(x, X, Y, y_part, node_init):
    n_batch, d_in = x.shape
    d_out = Y.shape[-1]
    info = plsc.get_sparse_core_info()
    nc, ns = info.num_cores, info.num_subcores
    nw = nc * ns
    b_per_w = n_batch // nw
    n_sub = b_per_w // _L
    n_deep = _DEPTH - _KTC

    def body(x_hbm, X_hbm, Y_hbm, ypart_hbm, node_hbm, out_hbm,
             x_v, y_v, xr_a, yr_a, xr_b, yr_b, node_r, node_nx, accm_r,
             sem_xa, sem_ya, sem_xb, sem_yb, sem_p1, sem_p2, sem_n, sem_w):
        wid = lax.axis_index("s") * nc + lax.axis_index("c")
        base = wid * b_per_w
        ids = lax.iota(jnp.int32, _L)
        bufs = [(xr_a, yr_a, sem_xa, sem_ya), (xr_b, yr_b, sem_xb, sem_yb)]

        pltpu.sync_copy(node_hbm.at[pl.ds(base, _L)], node_r)
        pltpu.async_copy(X_hbm.at[node_r], xr_a, sem_xa)
        pltpu.async_copy(Y_hbm.at[node_r], yr_a, sem_ya)

        def sub_chunk(sc_i, carry):
            tok0 = base + sc_i * _L

            @pl.when(sc_i > 0)
            def _():
                pltpu.make_async_copy(
                    y_v, out_hbm.at[pl.ds(tok0, _L)], sem_w).wait()

            @pl.when(sc_i == 0)
            def _():
                pltpu.async_copy(x_hbm.at[pl.ds(tok0, _L)], x_v, sem_p1)

            cp_yv = pltpu.async_copy(ypart_hbm.at[pl.ds(tok0, _L)], y_v,
                                     sem_p2)

            @pl.when(sc_i < n_sub - 1)
            def _():
                pltpu.async_copy(node_hbm.at[pl.ds(tok0 + _L, _L)],
                                 node_nx, sem_n)

            pltpu.make_async_copy(
                x_hbm.at[pl.ds(tok0, _L)], x_v, sem_p1).wait()
            for d in range(n_deep):
                xr_v, yr_v, sx, sy = bufs[d % 2]
                pltpu.make_async_copy(X_hbm.at[node_r], xr_v, sx).wait()

                def dot_body(i, accs):
                    sl = pl.ds(i * _L, _L)
                    return tuple(accs[t] + x_v[t, sl] * xr_v[t, sl]
                                 for t in range(_L))

                accs = lax.fori_loop(
                    0, d_in // _L, dot_body,
                    (jnp.zeros((_L,), jnp.float32),) * _L)
                for t in range(_L):
                    accm_r[pl.ds(t * _L, _L)] = accs[t]
                lam_vec = jnp.zeros((_L,), jnp.float32)
                for i in range(_L):
                    lam_vec = lam_vec + plsc.load_gather(
                        accm_r, [ids * _L + i])
                node = node_r[:]
                node_r[:] = node + node + 1 + (lam_vec > 0.0).astype(
                    jnp.int32)
                if d + 1 < n_deep:
                    nxr, nyr, nsx, nsy = bufs[(d + 1) % 2]
                    pltpu.async_copy(X_hbm.at[node_r], nxr, nsx)
                    pltpu.async_copy(Y_hbm.at[node_r], nyr, nsy)
                else:
                    @pl.when(sc_i < n_sub - 1)
                    def _():
                        pltpu.make_async_copy(
                            node_hbm.at[pl.ds(tok0 + _L, _L)],
                            node_nx, sem_n).wait()
                        node_r[:] = node_nx[:]
                        pltpu.async_copy(X_hbm.at[node_r], xr_a, sem_xa)
                        pltpu.async_copy(Y_hbm.at[node_r], yr_a, sem_ya)
                        pltpu.async_copy(x_hbm.at[pl.ds(tok0 + _L, _L)],
                                         x_v, sem_p1)

                lam_bs = [jnp.broadcast_to(lam_vec[t], (_L,))
                          for t in range(_L)]
                if d == 0:
                    cp_yv.wait()
                pltpu.make_async_copy(Y_hbm.at[node_r], yr_v, sy).wait()

                def acc_body(i, c2):
                    sl = pl.ds(i * _L, _L)
                    for t in range(_L):
                        y_v[t, sl] = y_v[t, sl] + lam_bs[t] * yr_v[t, sl]
                    return c2

                lax.fori_loop(0, d_out // _L, acc_body, 0)
            pltpu.async_copy(y_v, out_hbm.at[pl.ds(tok0, _L)], sem_w)
            return carry

        lax.fori_loop(0, n_sub, sub_chunk, 0)
        pltpu.make_async_copy(
            y_v, out_hbm.at[pl.ds(base + (n_sub - 1) * _L, _L)],
            sem_w).wait()

    fff = pl.kernel(
        body,
        out_type=jax.ShapeDtypeStruct((n_batch, d_out), jnp.float32),
        mesh=plsc.VectorSubcoreMesh(core_axis_name="c", subcore_axis_name="s"),
        compiler_params=pltpu.CompilerParams(needs_layout_passes=False),
        scratch_types=[
            pltpu.VMEM((_L, d_in), jnp.float32),
            pltpu.VMEM((_L, d_out), jnp.float32),
            pltpu.VMEM((_L, d_in), jnp.float32),
            pltpu.VMEM((_L, d_out), jnp.float32),
            pltpu.VMEM((_L, d_in), jnp.float32),
            pltpu.VMEM((_L, d_out), jnp.float32),
            pltpu.VMEM((_L,), jnp.int32),
            pltpu.VMEM((_L,), jnp.int32),
            pltpu.VMEM((_L * _L,), jnp.float32),
            pltpu.SemaphoreType.DMA,
            pltpu.SemaphoreType.DMA,
            pltpu.SemaphoreType.DMA,
            pltpu.SemaphoreType.DMA,
            pltpu.SemaphoreType.DMA,
            pltpu.SemaphoreType.DMA,
            pltpu.SemaphoreType.DMA,
            pltpu.SemaphoreType.DMA,
        ],
    )
    return fff(x, X, Y, y_part, node_init)


def kernel(x, X, Y):
    n_batch = x.shape[0]
    xt = jnp.swapaxes(X[:_NSH], 0, 1)
    ysh = Y[:_NSH]
    y_part, node3d = _tc_phase(x, xt, ysh)
    node_init = jnp.reshape(node3d, (n_batch,))
    return _sc_phase(x, X, Y, y_part, node_init)
